# baseline (speedup 1.0000x reference)
import functools

import jax
import jax.numpy as jnp
from jax import lax
from jax.experimental import pallas as pl
from jax.experimental.pallas import tpu as pltpu

_LANE = 128


def _extrema_scatter_kernel(x_ref, o_ref, *, pool_size: int, width: int,
                            region: int):
    xw = x_ref[...]
    ts, c_dim, pw = xw.shape
    lane = lax.broadcasted_iota(jnp.int32, xw.shape, 2)
    inwin = lane % width < pool_size
    aw = jnp.where(inwin, jnp.abs(xw), -1.0)
    m = jnp.max(aw, axis=-1, keepdims=True)
    cand = jnp.where(aw == m, lane, jnp.int32(pw))
    idx = jnp.min(cand, axis=-1, keepdims=True)
    col = lax.broadcasted_iota(jnp.int32, (1, pw), 1)
    hit = col == idx[:, 0, :]
    for c in range(1, c_dim):
        hit = hit | (col == idx[:, c, :])
    acc = jnp.where(hit, xw[:, 0, :], 0.0).astype(o_ref.dtype)
    if region > pw:
        acc = jnp.concatenate(
            [acc, jnp.zeros((ts, region - pw), o_ref.dtype)], axis=-1)
    o_ref[:, :region] = acc
    o_ref[:, region:] = jnp.zeros((ts, o_ref.shape[1] - region), o_ref.dtype)


def _extrema_pool_indices_2d(x, pool_size: int):
    N, C, H, W = x.shape
    HW = H * W
    itemsize = x.dtype.itemsize
    x3 = x.reshape(N, C, HW)

    pw = pool_size * W
    region = min(-(-pw // _LANE) * _LANE, C * HW)

    row = C * HW
    ts = max(8, min(N, (8 * 1024 * 1024) // (row * itemsize) // 8 * 8))

    out2 = pl.pallas_call(
        functools.partial(_extrema_scatter_kernel, pool_size=pool_size,
                          width=W, region=region),
        out_shape=jax.ShapeDtypeStruct((N, row), x.dtype),
        grid=(pl.cdiv(N, ts),),
        in_specs=[pl.BlockSpec((ts, C, pw), lambda i: (i, 0, 0))],
        out_specs=pl.BlockSpec((ts, row), lambda i: (i, 0)),
        compiler_params=pltpu.CompilerParams(
            dimension_semantics=("parallel",),
            vmem_limit_bytes=64 * 1024 * 1024,
        ),
        cost_estimate=pl.CostEstimate(
            flops=8 * N * C * pw + 2 * N * region,
            transcendentals=0,
            bytes_accessed=(N * row + N * C * pw) * itemsize,
        ),
    )(x3)
    return out2.reshape(N, C, H, W)


def kernel(x):
    return _extrema_pool_indices_2d(x, 4)


# speedup vs baseline: 1.4157x; 1.4157x over previous
import functools

import jax
import jax.numpy as jnp
from jax import lax
from jax.experimental import pallas as pl
from jax.experimental.pallas import tpu as pltpu

_LANE = 128


def _extrema_kernel(win_ref, o_hbm, zbuf, acc_ref, sems, *, pool_size: int,
                    width: int, region: int, zrows: int, n_chunks: int):
    n, c_dim, pp = win_ref.shape
    row = o_hbm.shape[1]

    zbuf[...] = jnp.zeros(zbuf.shape, zbuf.dtype)
    for k in range(n_chunks):
        pltpu.make_async_copy(
            zbuf,
            o_hbm.at[pl.ds(k * zrows, zrows), pl.ds(region, row - region)],
            sems.at[k],
        ).start()

    win = win_ref[...]
    awin = jnp.abs(win)
    jpos = lax.broadcasted_iota(jnp.int32, awin.shape, 2)
    m = jnp.max(awin, axis=-1, keepdims=True)
    cand = jnp.where(awin == m, jpos, jnp.int32(pp))
    idx = jnp.min(cand, axis=-1, keepdims=True)
    wcol = lax.broadcasted_iota(jnp.int32, (1, pp), 1)
    hit = wcol == idx[:, 0, :]
    for c in range(1, c_dim):
        hit = hit | (wcol == idx[:, c, :])
    strip = jnp.where(hit, win[:, 0, :], 0.0).astype(acc_ref.dtype)
    dcol = lax.broadcasted_iota(jnp.int32, (1, region), 1)
    acc = jnp.zeros((n, region), acc_ref.dtype)
    for j in range(pp):
        acc = jnp.where(dcol == (j // pool_size) * width + j % pool_size,
                        strip[:, j:j + 1], acc)
    acc_ref[...] = acc
    pltpu.make_async_copy(acc_ref, o_hbm.at[:, pl.ds(0, region)],
                          sems.at[n_chunks]).start()

    for k in range(n_chunks):
        pltpu.make_async_copy(
            zbuf,
            o_hbm.at[pl.ds(k * zrows, zrows), pl.ds(region, row - region)],
            sems.at[k],
        ).wait()
    pltpu.make_async_copy(acc_ref, o_hbm.at[:, pl.ds(0, region)],
                          sems.at[n_chunks]).wait()


def _extrema_pool_indices_2d(x, pool_size: int):
    N, C, H, W = x.shape
    HW = H * W
    pp = pool_size * pool_size
    row = C * HW
    win = x[:, :, :pool_size, :pool_size].reshape(N, C, pp)

    region = min(-(-(pool_size * W) // _LANE) * _LANE, row)
    zrows = min(256, N)
    n_chunks = N // zrows

    out2 = pl.pallas_call(
        functools.partial(_extrema_kernel, pool_size=pool_size, width=W,
                          region=region, zrows=zrows, n_chunks=n_chunks),
        out_shape=jax.ShapeDtypeStruct((N, row), x.dtype),
        in_specs=[pl.BlockSpec((N, C, pp), lambda: (0, 0, 0))],
        out_specs=pl.BlockSpec(memory_space=pl.ANY),
        scratch_shapes=[
            pltpu.VMEM((zrows, row - region), x.dtype),
            pltpu.VMEM((N, region), x.dtype),
            pltpu.SemaphoreType.DMA((n_chunks + 1,)),
        ],
        compiler_params=pltpu.CompilerParams(
            vmem_limit_bytes=64 * 1024 * 1024,
        ),
        cost_estimate=pl.CostEstimate(
            flops=8 * N * C * pp + 2 * N * region,
            transcendentals=0,
            bytes_accessed=(N * row + N * C * pp) * x.dtype.itemsize,
        ),
    )(win)
    return out2.reshape(N, C, H, W)


def kernel(x):
    return _extrema_pool_indices_2d(x, 4)
